# Initial kernel scaffold; baseline (speedup 1.0000x reference)
import jax
import jax.numpy as jnp
from jax.experimental import pallas as pl


def kernel(z, codebook):
    raise NotImplementedError("write your pallas kernel here")


# speedup vs baseline: 1.4086x; 1.4086x over previous
import functools

import jax
import jax.numpy as jnp
from jax import lax
from jax.experimental import pallas as pl
from jax.experimental.pallas import tpu as pltpu
from jax.experimental.pallas import tpu_sc as plsc

N_TOK = 16384
N_CODE = 8192
DIM = 64
BLK = 1024
GRID = N_TOK // BLK


HALF = N_CODE // 2


def _argmin_body(z_ref, cbt2_ref, zsq_ref, idx_ref, loss_ref, csq_ref, ids_ref):
    @pl.when(pl.program_id(0) == 0)
    def _():
        cbt2 = cbt2_ref[...]
        csq_ref[...] = jnp.sum(cbt2 * cbt2, axis=0, keepdims=True) * 0.25
        ids_ref[...] = lax.broadcasted_iota(
            jnp.int32, (1, HALF), 1).astype(jnp.float32)

    z = z_ref[...]
    zsq = zsq_ref[...]
    m2 = lax.dot_general(
        z, cbt2_ref[...], (((1,), (0,)), ((), ())),
        preferred_element_type=jnp.float32)
    d = (zsq + csq_ref[...]) - m2
    d1 = d[:, :HALF]
    d2 = d[:, HALF:]
    min1 = jnp.min(d1, axis=1, keepdims=True)
    min2 = jnp.min(d2, axis=1, keepdims=True)
    ids = ids_ref[...]
    fhalf = jnp.float32(HALF)
    i1f = jnp.min(jnp.where(d1 == min1, ids, fhalf), axis=1, keepdims=True)
    i2f = jnp.min(jnp.where(d2 == min2, ids, fhalf), axis=1, keepdims=True)
    v1b = min1.astype(jnp.bfloat16).astype(jnp.float32)
    keep1 = v1b <= min2
    idxf = jnp.where(keep1, i1f, i2f + fhalf)
    dmin = jnp.where(keep1, min1, min2)
    idx_ref[...] = idxf[:, 0].astype(jnp.int32)
    pid = pl.program_id(0)
    s = jnp.sum(dmin)
    prev = jnp.where(pid == 0, jnp.float32(0.0), loss_ref[0, 0])
    tot = prev + s
    loss_ref[0, 0] = jnp.where(
        pid == GRID - 1, tot * jnp.float32(1.0 / (N_TOK * DIM)), tot)


_argmin_call = pl.pallas_call(
    _argmin_body,
    grid=(GRID,),
    in_specs=[
        pl.BlockSpec((BLK, DIM), lambda i: (i, 0)),
        pl.BlockSpec((DIM, N_CODE), lambda i: (0, 0)),
        pl.BlockSpec((BLK, 1), lambda i: (i, 0)),
    ],
    out_specs=[
        pl.BlockSpec((BLK,), lambda i: (i,)),
        pl.BlockSpec(memory_space=pltpu.MemorySpace.SMEM),
    ],
    out_shape=[
        jax.ShapeDtypeStruct((N_TOK,), jnp.int32),
        jax.ShapeDtypeStruct((1, 1), jnp.float32),
    ],
    scratch_shapes=[
        pltpu.VMEM((1, N_CODE), jnp.float32),
        pltpu.VMEM((1, N_CODE // 2), jnp.float32),
    ],
)


@functools.cache
def _make_gather():
    info = plsc.get_sparse_core_info()
    nc, ns = info.num_cores, info.num_subcores
    nw = nc * ns
    b_per_w = N_TOK // nw
    mesh = plsc.VectorSubcoreMesh(core_axis_name="c", subcore_axis_name="s")

    @functools.partial(
        pl.kernel,
        mesh=mesh,
        out_type=jax.ShapeDtypeStruct((N_TOK, DIM), jnp.float32),
        scratch_types=[
            pltpu.VMEM((b_per_w,), jnp.int32),
            pltpu.VMEM((b_per_w, DIM), jnp.float32),
            pltpu.SemaphoreType.DMA,
        ],
        compiler_params=pltpu.CompilerParams(use_tc_tiling_on_sc=False),
    )
    def k(table_hbm, idx_hbm, out_hbm, idx_v, rows_v, sem):
        wid = lax.axis_index("s") * nc + lax.axis_index("c")
        base = wid * b_per_w
        pltpu.sync_copy(idx_hbm.at[pl.ds(base, b_per_w)], idx_v)
        pltpu.async_copy(table_hbm.at[idx_v], rows_v, sem).wait()
        pltpu.sync_copy(rows_v, out_hbm.at[pl.ds(base, b_per_w)])

    return k


def kernel(z, codebook):
    cbt2 = (codebook * 2.0).T
    zsq = jnp.sum(z ** 2, axis=1, keepdims=True)
    indices, loss2d = _argmin_call(z, cbt2, zsq)
    z_q = _make_gather()(codebook, indices)
    loss = loss2d.reshape(())
    return z_q, indices, loss, loss
